# baseline (speedup 1.0000x reference)
import functools

import jax
import jax.numpy as jnp
from jax import lax
from jax.experimental import pallas as pl
from jax.experimental.pallas import tpu as pltpu
from jax.experimental.pallas import tpu_sc as plsc

H = 2048
W = 2048
C = 8
B = 262144

NC = 2
NS = 16
NW = NC * NS
P = B // NW
N_BLK = 2048
NBLKS = P // N_BLK
CHUNKS = N_BLK // 16


def _sc_body(coords_hbm, b0_hbm, b1_hbm, table_hbm, out_hbm,
             coords_v, b0_v, b1_v, d0_v, d1_v,
             idx_a, idx_b, idx_c, idx_d,
             dst_a, dst_b, dst_c, dst_d,
             out_v, sem):
    wid = lax.axis_index("s") * NC + lax.axis_index("c")
    base_w = wid * P

    pltpu.sync_copy(b0_hbm, b0_v)
    pltpu.sync_copy(b1_hbm, b1_v)
    lanes = jnp.arange(16, dtype=jnp.int32)
    zeros16 = jnp.zeros((16,), dtype=jnp.int32)
    b0 = b0_v[...]
    b1 = b1_v[...]
    maxi = jnp.full((16,), H - 2, dtype=jnp.int32)
    fzero = jnp.zeros((16,), dtype=jnp.float32)

    for blk in range(NBLKS):
        base = base_w + blk * N_BLK
        pltpu.sync_copy(coords_hbm.at[pl.ds(base, N_BLK)], coords_v)

        def index_body(i, carry):
            i16 = i * 16
            row = i16 + lanes
            cr = plsc.load_gather(coords_v, [row, zeros16])
            cc = plsc.load_gather(coords_v, [row, zeros16 + 1])
            xr = jnp.maximum(cr - b0, fzero)
            xc = jnp.maximum(cc - b1, fzero)
            ri = jnp.minimum(xr.astype(jnp.int32), maxi)
            ci = jnp.minimum(xc.astype(jnp.int32), maxi)
            d0 = xr - ri.astype(jnp.float32)
            d1 = xc - ci.astype(jnp.float32)
            a = ri * W + ci
            idx_a[pl.ds(i16, 16)] = a
            idx_b[pl.ds(i16, 16)] = a + 1
            idx_c[pl.ds(i16, 16)] = a + W
            idx_d[pl.ds(i16, 16)] = a + (W + 1)
            d0_v[pl.ds(i16, 16)] = d0
            d1_v[pl.ds(i16, 16)] = d1
            return carry

        lax.fori_loop(0, CHUNKS, index_body, 0)

        cp_a = pltpu.make_async_copy(table_hbm.at[idx_a], dst_a, sem)
        cp_b = pltpu.make_async_copy(table_hbm.at[idx_b], dst_b, sem)
        cp_c = pltpu.make_async_copy(table_hbm.at[idx_c], dst_c, sem)
        cp_d = pltpu.make_async_copy(table_hbm.at[idx_d], dst_d, sem)
        cp_a.start()
        cp_b.start()
        cp_c.start()
        cp_d.start()
        cp_a.wait()
        cp_b.wait()
        cp_c.wait()
        cp_d.wait()

        def combine_body(i, carry):
            i16 = i * 16
            row = i16 + lanes
            d0 = d0_v[pl.ds(i16, 16)]
            d1 = d1_v[pl.ds(i16, 16)]
            w_tl = d0 * d1
            w_bl = d0 - w_tl
            w_tr = d1 - w_tl
            w_br = (1.0 - d0) - w_tr
            for ch in range(C):
                col = zeros16 + ch
                tl = plsc.load_gather(dst_a, [row, col])
                bl = plsc.load_gather(dst_b, [row, col])
                tr = plsc.load_gather(dst_c, [row, col])
                br = plsc.load_gather(dst_d, [row, col])
                acc = w_tl * tl + w_bl * bl + w_tr * tr + w_br * br
                plsc.store_scatter(out_v, [row, col], acc)
            return carry

        lax.fori_loop(0, CHUNKS, combine_body, 0)

        pltpu.sync_copy(out_v, out_hbm.at[pl.ds(base, N_BLK)])


@jax.jit
def _run(coords, b0_arr, b1_arr, table):
    mesh = plsc.VectorSubcoreMesh(
        core_axis_name="c", subcore_axis_name="s",
        num_cores=NC, num_subcores=NS)
    fn = pl.kernel(
        _sc_body,
        out_type=jax.ShapeDtypeStruct((B, C), jnp.float32),
        mesh=mesh,
        scratch_types=[
            pltpu.VMEM((N_BLK, 2), jnp.float32),
            pltpu.VMEM((16,), jnp.float32),
            pltpu.VMEM((16,), jnp.float32),
            pltpu.VMEM((N_BLK,), jnp.float32),
            pltpu.VMEM((N_BLK,), jnp.float32),
            pltpu.VMEM((N_BLK,), jnp.int32),
            pltpu.VMEM((N_BLK,), jnp.int32),
            pltpu.VMEM((N_BLK,), jnp.int32),
            pltpu.VMEM((N_BLK,), jnp.int32),
            pltpu.VMEM((N_BLK, C), jnp.float32),
            pltpu.VMEM((N_BLK, C), jnp.float32),
            pltpu.VMEM((N_BLK, C), jnp.float32),
            pltpu.VMEM((N_BLK, C), jnp.float32),
            pltpu.VMEM((N_BLK, C), jnp.float32),
            pltpu.SemaphoreType.DMA,
        ],
        compiler_params=pltpu.CompilerParams(
            needs_layout_passes=False, use_tc_tiling_on_sc=False),
    )
    return fn(coords, b0_arr, b1_arr, table)


def kernel(coords, visible, bias):
    table = visible.reshape(H * W, C)
    b0_arr = jnp.full((16,), bias[0], dtype=jnp.float32)
    b1_arr = jnp.full((16,), bias[1], dtype=jnp.float32)
    return _run(coords, b0_arr, b1_arr, table)


# speedup vs baseline: 12.1469x; 12.1469x over previous
import jax
import jax.numpy as jnp
from jax import lax
from jax.experimental import pallas as pl
from jax.experimental.pallas import tpu as pltpu
from jax.experimental.pallas import tpu_sc as plsc

H = 2048
W = 2048
C = 8
B = 262144

NC = 2
NS = 16
NW = NC * NS
P = B // NW
N_BLK = 512
NBLKS = P // N_BLK
CHUNKS = N_BLK // 16
NBUF = 4
ROWS_W = H // NW
RCH = W * C

_params = pltpu.CompilerParams(
    needs_layout_passes=False, use_tc_tiling_on_sc=False)


def _transpose_body(img_hbm, table_hbm, rowbuf, obuf, in_sems, out_sems):
    wid = lax.axis_index("s") * NC + lax.axis_index("c")
    row0 = wid * ROWS_W

    lanes = jnp.arange(16, dtype=jnp.int32)
    pat_lo = jnp.bitwise_and(lanes, 3) * 256 + lax.shift_right_logical(lanes, 2)
    pat_hi = pat_lo + 128
    himask = jnp.full((16,), -65536, dtype=jnp.int32)

    def start_in(r, k):
        pltpu.make_async_copy(
            img_hbm.at[jnp.minimum(r, H - 1)], rowbuf.at[k],
            in_sems.at[k]).start()

    def wait_in(k):
        pltpu.make_async_copy(
            img_hbm.at[0], rowbuf.at[k], in_sems.at[k]).wait()

    def start_out(r, k):
        pltpu.make_async_copy(
            obuf.at[k], table_hbm.at[pl.ds(r * (RCH // 2), RCH // 2)],
            out_sems.at[k]).start()

    def wait_out(k):
        pltpu.make_async_copy(
            obuf.at[0], table_hbm.at[pl.ds(0, RCH // 2)], out_sems.at[0],
        ).wait() if k == 0 else pltpu.make_async_copy(
            obuf.at[1], table_hbm.at[pl.ds(0, RCH // 2)], out_sems.at[1]).wait()

    for k in range(4):
        start_in(row0 + k, k)
    start_out(row0, 0)
    start_out(row0 + 1, 1)

    def quad_body(i, carry):
        r = row0 + 4 * i
        for k in range(4):
            ko = k % 2
            wait_in(k)
            wait_out(ko)

            @plsc.parallel_loop(0, W // 4, unroll=8)
            def shuffle(j):
                off = 4 * j + 896 * lax.shift_right_logical(j, 5)
                a = plsc.load_gather(rowbuf.at[k], [pat_lo + off])
                b = plsc.load_gather(rowbuf.at[k], [pat_hi + off])
                rnd = jnp.full((16,), 0x8000, jnp.int32)
                wa = lax.shift_right_logical(
                    plsc.bitcast(a, jnp.int32) + rnd, 16)
                wb = jnp.bitwise_and(plsc.bitcast(b, jnp.int32) + rnd, himask)
                obuf[ko, pl.ds(j * 16, 16)] = jnp.bitwise_or(wa, wb)

            start_in(r + k + 4, k)
            start_out(r + k, ko)
        return carry

    lax.fori_loop(0, ROWS_W // 4, quad_body, 0)
    wait_out(0)
    wait_out(1)
    for k in range(4):
        wait_in(k)


def _gather_body(coords_hbm, b0_hbm, b1_hbm, table_hbm, out_hbm,
                 coords_v, b0_v, b1_v, d0_v, d1_v, par_v, idx_v, dst_v,
                 out_v, sems):
    wid = lax.axis_index("s") * NC + lax.axis_index("c")
    base_w = wid * P

    pltpu.sync_copy(b0_hbm, b0_v)
    pltpu.sync_copy(b1_hbm, b1_v)
    lanes = jnp.arange(16, dtype=jnp.int32)
    zeros16 = jnp.zeros((16,), dtype=jnp.int32)
    b0 = b0_v[...]
    b1 = b1_v[...]
    maxi = jnp.full((16,), H - 2, dtype=jnp.int32)
    fzero = jnp.zeros((16,), dtype=jnp.float32)

    def stage(blk, buf):
        base = base_w + blk * N_BLK
        pltpu.sync_copy(
            coords_hbm.at[pl.ds(base // 128, N_BLK // 128)], coords_v.at[buf])

        @plsc.parallel_loop(0, CHUNKS, unroll=4)
        def index_body(i):
            i16 = i * 16
            blk128 = lax.shift_right_logical(i, 3)
            l0 = (i16 % 128)
            cr = coords_v[buf, blk128, 0, pl.ds(l0, 16)]
            cc = coords_v[buf, blk128, 1, pl.ds(l0, 16)]
            xr = jnp.maximum(cr - b0, fzero)
            xc = jnp.maximum(cc - b1, fzero)
            ri = jnp.minimum(xr.astype(jnp.int32), maxi)
            ci = jnp.minimum(xc.astype(jnp.int32), maxi)
            d0 = xr - ri.astype(jnp.float32)
            d1 = xc - ci.astype(jnp.float32)
            a = ri * W + ci
            g = lax.shift_right_logical(a, 1)
            idx_v[buf, 0, pl.ds(i16, 16)] = g
            idx_v[buf, 1, pl.ds(i16, 16)] = g + 1
            idx_v[buf, 2, pl.ds(i16, 16)] = g + W // 2
            idx_v[buf, 3, pl.ds(i16, 16)] = jnp.minimum(
                g + (W // 2 + 1), jnp.full((16,), H * W // 2 - 1, jnp.int32))
            par_v[buf, pl.ds(i16, 16)] = jnp.bitwise_and(a, 1)
            d0_v[buf, pl.ds(i16, 16)] = d0
            d1_v[buf, pl.ds(i16, 16)] = d1

        cps = [
            pltpu.make_async_copy(
                table_hbm.at[idx_v.at[buf, k]], dst_v.at[buf, k], sems.at[buf])
            for k in range(4)
        ]
        for cp in cps:
            cp.start()
        return cps

    def combine(blk, buf):
        base = base_w + blk * N_BLK

        @plsc.parallel_loop(0, CHUNKS, unroll=2)
        def combine_body(i):
            i16 = i * 16
            row = i16 + lanes
            d0 = d0_v[buf, pl.ds(i16, 16)]
            d1 = d1_v[buf, pl.ds(i16, 16)]
            w_tl = d0 * d1
            w_bl = d0 - w_tl
            w_tr = d1 - w_tl
            w_br = (1.0 - d0) - w_tr
            himask = jnp.full((16,), -65536, jnp.int32)
            par = par_v[buf, pl.ds(i16, 16)]
            par4 = lax.shift_left(par, 2)
            inv4 = 4 - par4
            two = jnp.full((16,), 2, jnp.int32)
            dst3 = dst_v.at[buf]

            def halves(w):
                lo = plsc.bitcast(lax.shift_left(w, 16), jnp.float32)
                hi = plsc.bitcast(jnp.bitwise_and(w, himask), jnp.float32)
                return lo, hi

            for wd in range(C // 2):
                ctl = par4 + wd
                cbl = inv4 + wd
                tl = halves(plsc.load_gather(dst3, [zeros16, row, ctl]))
                bl = halves(plsc.load_gather(dst3, [par, row, cbl]))
                tr = halves(plsc.load_gather(dst3, [two, row, ctl]))
                br = halves(plsc.load_gather(dst3, [two + par, row, cbl]))
                for half in range(2):
                    acc = (w_tl * tl[half] + w_bl * bl[half]
                           + w_tr * tr[half] + w_br * br[half])
                    out_v[buf, 2 * wd + half, pl.ds(i16, 16)] = acc

        pltpu.sync_copy(out_v.at[buf], out_hbm.at[:, pl.ds(base, N_BLK)])

    inflight = [stage(b, b % NBUF) for b in range(NBUF - 1)]
    for blk in range(NBLKS):
        nxt = blk + NBUF - 1
        if nxt < NBLKS:
            inflight.append(stage(nxt, nxt % NBUF))
        for cp in inflight.pop(0):
            cp.wait()
        combine(blk, blk % NBUF)


@jax.jit
def _run(coords, b0_arr, b1_arr, img):
    mesh = plsc.VectorSubcoreMesh(
        core_axis_name="c", subcore_axis_name="s",
        num_cores=NC, num_subcores=NS)
    transpose_fn = pl.kernel(
        _transpose_body,
        out_type=jax.ShapeDtypeStruct((H * W * C // 2,), jnp.int32),
        mesh=mesh,
        scratch_types=[
            pltpu.VMEM((4, RCH), jnp.float32),
            pltpu.VMEM((2, RCH // 2), jnp.int32),
            pltpu.SemaphoreType.DMA((4,)),
            pltpu.SemaphoreType.DMA((2,)),
        ],
        compiler_params=_params,
    )
    table = transpose_fn(img).reshape(H * W // 2, C)
    gather_fn = pl.kernel(
        _gather_body,
        out_type=jax.ShapeDtypeStruct((C, B), jnp.float32),
        mesh=mesh,
        scratch_types=[
            pltpu.VMEM((NBUF, N_BLK // 128, 2, 128), jnp.float32),
            pltpu.VMEM((16,), jnp.float32),
            pltpu.VMEM((16,), jnp.float32),
            pltpu.VMEM((NBUF, N_BLK), jnp.float32),
            pltpu.VMEM((NBUF, N_BLK), jnp.float32),
            pltpu.VMEM((NBUF, N_BLK), jnp.int32),
            pltpu.VMEM((NBUF, 4, N_BLK), jnp.int32),
            pltpu.VMEM((NBUF, 4, N_BLK, C), jnp.int32),
            pltpu.VMEM((NBUF, C, N_BLK), jnp.float32),
            pltpu.SemaphoreType.DMA((NBUF,)),
        ],
        compiler_params=_params,
    )
    return gather_fn(coords, b0_arr, b1_arr, table)


def kernel(coords, visible, bias):
    img = visible.reshape(H, W // 128, 128, C).transpose(0, 1, 3, 2)
    img = img.reshape(H, RCH)
    coords_b = coords.reshape(B // 128, 128, 2).transpose(0, 2, 1)
    b0_arr = jnp.full((16,), bias[0], dtype=jnp.float32)
    b1_arr = jnp.full((16,), bias[1], dtype=jnp.float32)
    out_cm = _run(coords_b, b0_arr, b1_arr, img)
    return out_cm.T
